# baseline (speedup 1.0000x reference)
import functools

import jax
import jax.numpy as jnp
from jax import lax
from jax.experimental import pallas as pl
from jax.experimental.pallas import tpu as pltpu
from jax.experimental.pallas import tpu_sc as plsc

V = 100000
D = 64
DW = 128
B = 16384
L = 50
K = 5
ALPHA = 0.5

NC = 2
NS = 16
NW = NC * NS
BPW = B // NW
CB = 8
NCHUNK = BPW // CB

LP = 64
KP = 16


def _perm(vec, idx):
    dnums = lax.GatherDimensionNumbers(
        offset_dims=(), collapsed_slice_dims=(0,), start_index_map=(0,))
    return lax.gather(vec, idx[:, None], dnums, slice_sizes=(1,),
                      mode=lax.GatherScatterMode.PROMISE_IN_BOUNDS)


def _sc_body(tgt_ref, ctx_ref, negi_ref, tbl_ref,
             pos_out, neg_out, *scr):
    bufs = (scr[0:6], scr[6:12])
    pos_sv, neg_sv = scr[12], scr[13]
    sems = (scr[14], scr[15])

    wid = lax.axis_index("s") * NC + lax.axis_index("c")
    row0 = wid * BPW

    def copies(p):
        tgt_i, ctx_i, neg_i, orp, ctxr, negr = bufs[p]
        return (
            (tbl_ref.at[tgt_i], orp),
            (tbl_ref.at[ctx_i], ctxr),
            (tbl_ref.at[neg_i], negr),
        )

    def stage_and_fire(ci, p):
        nbase = row0 + ci * CB
        tgt_i, ctx_i, neg_i = bufs[p][0], bufs[p][1], bufs[p][2]
        pltpu.sync_copy(tgt_ref.at[pl.ds(nbase, CB)], tgt_i)
        pltpu.sync_copy(ctx_ref.at[pl.ds(nbase * L, CB * L)], ctx_i)
        pltpu.sync_copy(negi_ref.at[pl.ds(nbase * K, CB * K)], neg_i)
        for src, dst in copies(p):
            pltpu.async_copy(src, dst, sems[p])

    def drain(p):
        for src, dst in copies(p):
            pltpu.make_async_copy(src, dst, sems[p]).wait()

    def compute(ci, p):
        base = row0 + ci * CB
        orp, ctxr, negr = bufs[p][3], bufs[p][4], bufs[p][5]
        iota16 = jnp.arange(16, dtype=jnp.int32)
        bfly = [iota16 ^ 1, iota16 ^ 2, iota16 ^ 4, iota16 ^ 8]
        masks = [iota16 == j for j in range(16)]

        def b_body(b, _):
            og = [orp[b, pl.ds(D + g * 16, 16)] for g in range(4)]

            def dot_all(rows_ref, r):
                t = og[0] * rows_ref[r, pl.ds(0, 16)]
                for g in range(1, 4):
                    t = t + og[g] * rows_ref[r, pl.ds(g * 16, 16)]
                for s in bfly:
                    t = t + _perm(t, s)
                return t

            for g in range(4):
                nl = min(16, L - g * 16)
                sv = jnp.zeros((16,), jnp.float32)
                for j in range(nl):
                    t = dot_all(ctxr, b * L + g * 16 + j)
                    sv = jnp.where(masks[j], t, sv)
                pos_sv[pl.ds(b * LP + g * 16, 16)] = sv

            sv = jnp.zeros((16,), jnp.float32)
            for k in range(K):
                t = dot_all(negr, b * K + k)
                sv = jnp.where(masks[k], t, sv)
            neg_sv[pl.ds(b * KP, 16)] = sv
            return _

        lax.fori_loop(0, CB, b_body, None)
        pltpu.sync_copy(pos_sv, pos_out.at[pl.ds(base * LP, CB * LP)])
        pltpu.sync_copy(neg_sv, neg_out.at[pl.ds(base * KP, CB * KP)])

    stage_and_fire(0, 0)

    def outer_body(co, carry):
        for p in range(2):
            ci = co * 2 + p

            @pl.when(ci + 1 < NCHUNK)
            def _():
                stage_and_fire(ci + 1, 1 - p)

            drain(p)
            compute(ci, p)
        return carry

    lax.fori_loop(0, NCHUNK // 2, outer_body, None)


def _buf_set():
    return [
        pltpu.VMEM((CB,), jnp.int32),
        pltpu.VMEM((CB * L,), jnp.int32),
        pltpu.VMEM((CB * K,), jnp.int32),
        pltpu.VMEM((CB, DW), jnp.float32),
        pltpu.VMEM((CB * L, DW), jnp.float32),
        pltpu.VMEM((CB * K, DW), jnp.float32),
    ]


_sc_scores = functools.partial(
    pl.kernel,
    out_type=(
        jax.ShapeDtypeStruct((B * LP,), jnp.float32),
        jax.ShapeDtypeStruct((B * KP,), jnp.float32),
    ),
    mesh=plsc.VectorSubcoreMesh(
        core_axis_name="c", subcore_axis_name="s",
        num_cores=NC, num_subcores=NS),
    compiler_params=pltpu.CompilerParams(
        needs_layout_passes=False, use_tc_tiling_on_sc=False),
    scratch_types=_buf_set() + _buf_set() + [
        pltpu.VMEM((CB * LP,), jnp.float32),
        pltpu.VMEM((CB * KP,), jnp.float32),
        pltpu.SemaphoreType.DMA,
        pltpu.SemaphoreType.DMA,
    ],
)(_sc_body)


def _loss_body(pos_ref, neg_ref, out_ref):
    pos = pos_ref[...]
    lane_p = lax.broadcasted_iota(jnp.int32, (B * LP,), 0) % LP
    pt = -jnp.log(jax.nn.sigmoid(pos) + 1e-6)
    psum = jnp.sum(jnp.where(lane_p < L, pt, 0.0))

    neg = neg_ref[...]
    lane_n = lax.broadcasted_iota(jnp.int32, (B * KP,), 0) % KP
    nt = -jnp.log(jax.nn.sigmoid(-neg) + 1e-6)
    nsum = jnp.sum(jnp.where(lane_n < K, nt, 0.0))

    out_ref[0, 0] = psum / (B * float(L)) + nsum / float(K)


_loss_tc = pl.pallas_call(
    _loss_body,
    out_shape=jax.ShapeDtypeStruct((1, 1), jnp.float32),
    out_specs=pl.BlockSpec(memory_space=pltpu.SMEM),
)


def kernel(target_tokens, context_tokens, neg_idx, W_fwd, W_rev, W_iso,
           token_frequencies):
    tgt = target_tokens.astype(jnp.int32)
    ctx = context_tokens.astype(jnp.int32).reshape(B * L)
    neg = neg_idx.astype(jnp.int32).reshape(B * K)
    scal = 1.0 / (1.0 + jnp.log(token_frequencies + 1e-6))
    comb = ALPHA * W_fwd + (1.0 - ALPHA) * W_rev + scal[:, None] * W_iso
    tbl = jnp.concatenate([W_fwd, comb], axis=1)
    pos_s, neg_s = _sc_scores(tgt, ctx, neg, tbl)
    return _loss_tc(pos_s, neg_s)[0, 0]


# speedup vs baseline: 1.3216x; 1.3216x over previous
import functools

import jax
import jax.numpy as jnp
from jax import lax
from jax.experimental import pallas as pl
from jax.experimental.pallas import tpu as pltpu
from jax.experimental.pallas import tpu_sc as plsc

V = 100000
D = 64
B = 16384
L = 50
K = 5
ALPHA = 0.5

NC = 2
NS = 16
NW = NC * NS
BPW = B // NW
CB = 16
NCHUNK = BPW // CB

LP = 64
KP = 16


def _perm(vec, idx):
    dnums = lax.GatherDimensionNumbers(
        offset_dims=(), collapsed_slice_dims=(0,), start_index_map=(0,))
    return lax.gather(vec, idx[:, None], dnums, slice_sizes=(1,),
                      mode=lax.GatherScatterMode.PROMISE_IN_BOUNDS)


def _bcast_lane(vec, lane):
    return _perm(vec, jnp.full((16,), lane, dtype=jnp.int32))


def _sc_body(tgt_ref, ctx_ref, negi_ref, wf_ref, wr_ref, wiso_ref, scal_ref,
             pos_out, neg_out, *scr):
    bufs = (scr[0:9], scr[9:18])
    pos_sv, neg_sv = scr[18], scr[19]
    gsem = (scr[20], scr[21])
    isem = (scr[22], scr[23])

    wid = lax.axis_index("s") * NC + lax.axis_index("c")
    row0 = wid * BPW

    def idx_copies(ci, p):
        nbase = row0 + ci * CB
        tgt_i, ctx_i, neg_i = bufs[p][0], bufs[p][1], bufs[p][2]
        return (
            (tgt_ref.at[pl.ds(nbase, CB)], tgt_i),
            (ctx_ref.at[pl.ds(nbase * L, CB * L)], ctx_i),
            (negi_ref.at[pl.ds(nbase * K, CB * K)], neg_i),
        )

    def gathers(p):
        tgt_i, ctx_i, neg_i, fwd, rev, iso, scal, ctxr, negr = bufs[p]
        return (
            (wf_ref.at[tgt_i], fwd),
            (wr_ref.at[tgt_i], rev),
            (wiso_ref.at[tgt_i], iso),
            (scal_ref.at[tgt_i], scal),
            (wf_ref.at[ctx_i], ctxr),
            (wf_ref.at[neg_i], negr),
        )

    def stage_idx(ci, p):
        for src, dst in idx_copies(ci, p):
            pltpu.async_copy(src, dst, isem[p])

    def wait_idx(ci, p):
        for src, dst in idx_copies(ci, p):
            pltpu.make_async_copy(src, dst, isem[p]).wait()

    def fire_gathers(p):
        for src, dst in gathers(p):
            pltpu.async_copy(src, dst, gsem[p])

    def drain_gathers(p):
        for src, dst in gathers(p):
            pltpu.make_async_copy(src, dst, gsem[p]).wait()

    def compute(ci, p):
        base = row0 + ci * CB
        _, _, _, fwd, rev, iso, scal, ctxr, negr = bufs[p]
        sc_all = scal[pl.ds(0, 16)]
        iota16 = jnp.arange(16, dtype=jnp.int32)
        bfly = [iota16 ^ 1, iota16 ^ 2, iota16 ^ 4, iota16 ^ 8]
        masks = [iota16 == j for j in range(16)]

        def b_body(b, _):
            scb = _bcast_lane(sc_all, b)
            og = []
            for g in range(4):
                f = fwd[b, pl.ds(g * 16, 16)]
                r = rev[b, pl.ds(g * 16, 16)]
                s = iso[b, pl.ds(g * 16, 16)]
                og.append(f * ALPHA + r * (1.0 - ALPHA) + s * scb)

            def dot_all(rows_ref, r):
                t = og[0] * rows_ref[r, pl.ds(0, 16)]
                for g in range(1, 4):
                    t = t + og[g] * rows_ref[r, pl.ds(g * 16, 16)]
                for s in bfly:
                    t = t + _perm(t, s)
                return t

            for g in range(4):
                nl = min(16, L - g * 16)
                sv = jnp.zeros((16,), jnp.float32)
                for j in range(nl):
                    t = dot_all(ctxr, b * L + g * 16 + j)
                    sv = jnp.where(masks[j], t, sv)
                pos_sv[pl.ds(b * LP + g * 16, 16)] = sv

            sv = jnp.zeros((16,), jnp.float32)
            for k in range(K):
                t = dot_all(negr, b * K + k)
                sv = jnp.where(masks[k], t, sv)
            neg_sv[pl.ds(b * KP, 16)] = sv
            return _

        lax.fori_loop(0, CB, b_body, None)
        pltpu.sync_copy(pos_sv, pos_out.at[pl.ds(base * LP, CB * LP)])
        pltpu.sync_copy(neg_sv, neg_out.at[pl.ds(base * KP, CB * KP)])

    stage_idx(0, 0)
    wait_idx(0, 0)
    fire_gathers(0)
    stage_idx(1, 1)

    def outer_body(co, carry):
        for p in range(2):
            ci = co * 2 + p

            drain_gathers(p)

            @pl.when(ci + 1 < NCHUNK)
            def _():
                wait_idx(ci + 1, 1 - p)
                fire_gathers(1 - p)

            @pl.when(ci + 2 < NCHUNK)
            def _():
                stage_idx(ci + 2, p)

            compute(ci, p)
        return carry

    lax.fori_loop(0, NCHUNK // 2, outer_body, None)


def _buf_set():
    return [
        pltpu.VMEM((CB,), jnp.int32),
        pltpu.VMEM((CB * L,), jnp.int32),
        pltpu.VMEM((CB * K,), jnp.int32),
        pltpu.VMEM((CB, D), jnp.float32),
        pltpu.VMEM((CB, D), jnp.float32),
        pltpu.VMEM((CB, D), jnp.float32),
        pltpu.VMEM((CB,), jnp.float32),
        pltpu.VMEM((CB * L, D), jnp.float32),
        pltpu.VMEM((CB * K, D), jnp.float32),
    ]


_sc_scores = functools.partial(
    pl.kernel,
    out_type=(
        jax.ShapeDtypeStruct((B * LP,), jnp.float32),
        jax.ShapeDtypeStruct((B * KP,), jnp.float32),
    ),
    mesh=plsc.VectorSubcoreMesh(
        core_axis_name="c", subcore_axis_name="s",
        num_cores=NC, num_subcores=NS),
    compiler_params=pltpu.CompilerParams(
        needs_layout_passes=False, use_tc_tiling_on_sc=False),
    scratch_types=_buf_set() + _buf_set() + [
        pltpu.VMEM((CB * LP,), jnp.float32),
        pltpu.VMEM((CB * KP,), jnp.float32),
        pltpu.SemaphoreType.DMA,
        pltpu.SemaphoreType.DMA,
        pltpu.SemaphoreType.DMA,
        pltpu.SemaphoreType.DMA,
    ],
)(_sc_body)


def _loss_body(pos_ref, neg_ref, out_ref):
    pos = pos_ref[...]
    lane_p = lax.broadcasted_iota(jnp.int32, (B * LP,), 0) % LP
    pt = -jnp.log(jax.nn.sigmoid(pos) + 1e-6)
    psum = jnp.sum(jnp.where(lane_p < L, pt, 0.0))

    neg = neg_ref[...]
    lane_n = lax.broadcasted_iota(jnp.int32, (B * KP,), 0) % KP
    nt = -jnp.log(jax.nn.sigmoid(-neg) + 1e-6)
    nsum = jnp.sum(jnp.where(lane_n < K, nt, 0.0))

    out_ref[0, 0] = psum / (B * float(L)) + nsum / float(K)


_loss_tc = pl.pallas_call(
    _loss_body,
    out_shape=jax.ShapeDtypeStruct((1, 1), jnp.float32),
    out_specs=pl.BlockSpec(memory_space=pltpu.SMEM),
)


def kernel(target_tokens, context_tokens, neg_idx, W_fwd, W_rev, W_iso,
           token_frequencies):
    tgt = target_tokens.astype(jnp.int32)
    ctx = context_tokens.astype(jnp.int32).reshape(B * L)
    neg = neg_idx.astype(jnp.int32).reshape(B * K)
    scal = 1.0 / (1.0 + jnp.log(token_frequencies + 1e-6))
    pos_s, neg_s = _sc_scores(tgt, ctx, neg, W_fwd, W_rev, W_iso, scal)
    return _loss_tc(pos_s, neg_s)[0, 0]
